# Initial kernel scaffold; baseline (speedup 1.0000x reference)
import jax
import jax.numpy as jnp
from jax.experimental import pallas as pl


def kernel(x, edge_index, edge_weight, W_rel1, b_rel1, W_root1, W_rel2, b_rel2, W_root2, W_rel3, b_rel3, W_root3, W_lin, b_lin):
    raise NotImplementedError("write your pallas kernel here")


# speedup vs baseline: 1.0004x; 1.0004x over previous
import functools
import jax
import jax.numpy as jnp
from jax.experimental import pallas as pl
from jax.experimental.pallas import tpu as pltpu

N = 50000
H = 128
BLK = 2000


def _layer_body(agg_ref, x_ref, rcp_ref, wrel_ref, wroot_ref, b_ref, out_ref):
    agg = agg_ref[...] * rcp_ref[...]
    y = jnp.dot(agg, wrel_ref[...], preferred_element_type=jnp.float32)
    y += jnp.dot(x_ref[...], wroot_ref[...], preferred_element_type=jnp.float32)
    out_ref[...] = jnp.maximum(y + b_ref[...], 0.0)


def _layer_matmul(agg, x, rcp, W_rel, W_root, b):
    f_in = x.shape[1]
    grid = (N // BLK,)
    return pl.pallas_call(
        _layer_body,
        grid=grid,
        in_specs=[
            pl.BlockSpec((BLK, f_in), lambda i: (i, 0)),
            pl.BlockSpec((BLK, f_in), lambda i: (i, 0)),
            pl.BlockSpec((BLK, 1), lambda i: (i, 0)),
            pl.BlockSpec((f_in, H), lambda i: (0, 0)),
            pl.BlockSpec((f_in, H), lambda i: (0, 0)),
            pl.BlockSpec((1, H), lambda i: (0, 0)),
        ],
        out_specs=pl.BlockSpec((BLK, H), lambda i: (i, 0)),
        out_shape=jax.ShapeDtypeStruct((N, H), jnp.float32),
    )(agg, x, rcp, W_rel.T, W_root.T, b[None, :])


def _final_body(x1_ref, x2_ref, x3_ref, w1_ref, w2_ref, w3_ref, b_ref, out_ref):
    y = jnp.dot(x1_ref[...], w1_ref[...], preferred_element_type=jnp.float32)
    y += jnp.dot(x2_ref[...], w2_ref[...], preferred_element_type=jnp.float32)
    y += jnp.dot(x3_ref[...], w3_ref[...], preferred_element_type=jnp.float32)
    out_ref[...] = y + b_ref[...]


def _final_matmul(x1, x2, x3, W_lin, b_lin):
    w1 = W_lin[:, :H].T
    w2 = W_lin[:, H:2 * H].T
    w3 = W_lin[:, 2 * H:].T
    grid = (N // BLK,)
    spec = pl.BlockSpec((BLK, H), lambda i: (i, 0))
    wspec = pl.BlockSpec((H, H), lambda i: (0, 0))
    return pl.pallas_call(
        _final_body,
        grid=grid,
        in_specs=[spec, spec, spec, wspec, wspec, wspec,
                  pl.BlockSpec((1, H), lambda i: (0, 0))],
        out_specs=spec,
        out_shape=jax.ShapeDtypeStruct((N, H), jnp.float32),
    )(x1, x2, x3, w1, w2, w3, b_lin[None, :])


def kernel(x, edge_index, edge_weight, W_rel1, b_rel1, W_root1, W_rel2, b_rel2, W_root2, W_rel3, b_rel3, W_root3, W_lin, b_lin):
    src = edge_index[0]
    dst = edge_index[1]
    cnt = jax.ops.segment_sum(jnp.ones((src.shape[0],), jnp.float32), dst,
                              num_segments=N)
    rcp = (1.0 / jnp.clip(cnt, 1.0, None))[:, None]

    def agg_of(v):
        msg = v[src] * edge_weight[:, None]
        return jax.ops.segment_sum(msg, dst, num_segments=N)

    x_pad = jnp.pad(x, ((0, 0), (0, 2)))
    Wr1 = jnp.pad(W_rel1, ((0, 0), (0, 2)))
    Wt1 = jnp.pad(W_root1, ((0, 0), (0, 2)))
    x1 = _layer_matmul(agg_of(x_pad), x_pad, rcp, Wr1, Wt1, b_rel1)
    x2 = _layer_matmul(agg_of(x1), x1, rcp, W_rel2, W_root2, b_rel2)
    x3 = _layer_matmul(agg_of(x2), x2, rcp, W_rel3, W_root3, b_rel3)
    return _final_matmul(x1, x2, x3, W_lin, b_lin)
